# baseline (speedup 1.0000x reference)
import functools

import jax
import jax.numpy as jnp
from jax import lax
from jax.experimental import pallas as pl
from jax.experimental.pallas import tpu as pltpu
from jax.experimental.pallas import tpu_sc as plsc

_NC = 2
_NS = 16
_NW = _NC * _NS
_L = 16
_TR = 8


def _const_vec(iota, values):
    res = iota * 0 + values[0]
    for k in range(1, _L):
        d = values[k] - values[k - 1]
        if d:
            res = res + jnp.where(iota >= k, d, 0)
    return res


@functools.lru_cache(maxsize=None)
def _build(n, feat, emb, dtype_name):
    dtype = jnp.dtype(dtype_name)
    out_w = feat + emb
    assert out_w % _TR == 0
    ntr = out_w // _TR
    rows_per_w = n // _NW
    chunk = min(128, rows_per_w)
    nchunk = rows_per_w // chunk
    nbuf = 2
    mesh = plsc.VectorSubcoreMesh(core_axis_name="c", subcore_axis_name="s")

    @functools.partial(
        pl.kernel,
        mesh=mesh,
        compiler_params=pltpu.CompilerParams(needs_layout_passes=False),
        out_type=jax.ShapeDtypeStruct((out_w, n), dtype),
        scratch_types=[
            *[pltpu.VMEM((chunk,), jnp.int32) for _ in range(nbuf)],
            *[pltpu.VMEM((chunk,), jnp.int32) for _ in range(nbuf)],
            *[pltpu.VMEM((feat, chunk), dtype) for _ in range(nbuf)],
            *[pltpu.VMEM((chunk, emb), dtype) for _ in range(nbuf)],
            *[pltpu.VMEM((chunk, emb), dtype) for _ in range(nbuf)],
            *[pltpu.VMEM((ntr, _TR, chunk), dtype) for _ in range(nbuf)],
            *[pltpu.SemaphoreType.DMA for _ in range(nbuf)],
            *[pltpu.SemaphoreType.DMA for _ in range(nbuf)],
            *[pltpu.SemaphoreType.DMA for _ in range(nbuf)],
            *[pltpu.SemaphoreType.DMA for _ in range(nbuf)],
        ],
    )
    def sc_kernel(lanesT_hbm, rid_hbm, lid_hbm, rtab_hbm, ltab_hbm, outT_hbm,
                  *scr):
        rid_v = scr[0:nbuf]
        lid_v = scr[nbuf:2 * nbuf]
        feat_v = scr[2 * nbuf:3 * nbuf]
        rrow_v = scr[3 * nbuf:4 * nbuf]
        lrow_v = scr[4 * nbuf:5 * nbuf]
        stg_v = scr[5 * nbuf:6 * nbuf]
        sem_r = scr[6 * nbuf:7 * nbuf]
        sem_l = scr[7 * nbuf:8 * nbuf]
        sem_o = scr[8 * nbuf:9 * nbuf]
        sem_f = scr[9 * nbuf:10 * nbuf]

        wid = lax.axis_index("s") * _NC + lax.axis_index("c")
        base = wid * rows_per_w

        iota = lax.iota(jnp.int32, _L)
        tr_vecs, sr_vecs = [], []
        for s in range(emb // _L):
            xs = [feat + s * _L + k for k in range(_L)]
            tr_vecs.append(_const_vec(iota, [x // _TR for x in xs]))
            sr_vecs.append(_const_vec(iota, [x % _TR for x in xs]))

        def start_fetch(c):
            b = c % nbuf
            lane0 = base + c * chunk
            pltpu.sync_copy(rid_hbm.at[pl.ds(lane0, chunk)], rid_v[b])
            pltpu.sync_copy(lid_hbm.at[pl.ds(lane0, chunk)], lid_v[b])
            cp_r = pltpu.async_copy(rtab_hbm.at[rid_v[b]], rrow_v[b], sem_r[b])
            cp_l = pltpu.async_copy(ltab_hbm.at[lid_v[b]], lrow_v[b], sem_l[b])
            cp_f = pltpu.async_copy(lanesT_hbm.at[:, pl.ds(lane0, chunk)],
                                    feat_v[b], sem_f[b])
            return cp_r, cp_l, cp_f

        fetches = {0: start_fetch(0)}
        out_cps = {}
        for c in range(nchunk):
            b = c % nbuf
            lane0 = base + c * chunk
            if c + 1 < nchunk:
                fetches[c + 1] = start_fetch(c + 1)
            if c - nbuf >= 0:
                for cp in out_cps.pop(c - nbuf):
                    cp.wait()
            cp_r, cp_l, cp_f = fetches.pop(c)
            cp_f.wait()
            for f in range(feat):
                for g in range(chunk // _L):
                    stg_v[b][0, f, pl.ds(g * _L, _L)] = (
                        feat_v[b][f, pl.ds(g * _L, _L)])
            cp_r.wait()
            cp_l.wait()

            @plsc.parallel_loop(0, chunk, carry=iota * 0)
            def add_lane(j, jv, _b=b):
                for s in range(emb // _L):
                    vals = (rrow_v[_b][j, pl.ds(s * _L, _L)]
                            + lrow_v[_b][j, pl.ds(s * _L, _L)])
                    plsc.store_scatter(stg_v[_b],
                                       [tr_vecs[s], sr_vecs[s], jv], vals)
                return jv + 1

            cps = []
            for r in range(ntr):
                cps.append(pltpu.async_copy(
                    stg_v[b].at[r],
                    outT_hbm.at[pl.ds(r * _TR, _TR), pl.ds(lane0, chunk)],
                    sem_o[b]))
            out_cps[c] = cps
        for c in sorted(out_cps):
            for cp in out_cps.pop(c):
                cp.wait()

    return sc_kernel


def kernel(lanes, road_id, lane_id, road_table, lane_table):
    n, feat = lanes.shape
    emb = road_table.shape[1]
    fn = _build(n, feat, emb, str(road_table.dtype))
    outT = fn(lanes.T,
              road_id.astype(jnp.int32),
              lane_id.astype(jnp.int32),
              road_table,
              lane_table)
    return outT.T
